# Initial kernel scaffold; baseline (speedup 1.0000x reference)
import functools

import jax
import jax.numpy as jnp
from jax.experimental import pallas as pl

KH, KW = 32, 32
STRIDE = 4
PAD = 10
H, W, C = 224, 224, 3
DIM = KH * KW * C
N_MEM = 4096
OH = (H + 2 * PAD - KH) // STRIDE + 1
OW = (W + 2 * PAD - KW) // STRIDE + 1
Q = OH * OW
U = (H + 2 * PAD) // STRIDE
AR = KH // STRIDE
AS = KW // STRIDE

TK = 512


def _unfold_kernel(img_ref, q_ref, qsq_ref):
    i = pl.program_id(0)
    blk = img_ref[pl.ds(i, AR)]
    X = blk.reshape(AR, C, STRIDE, U, STRIDE)
    parts = [X[:, :, :, a : a + OW, :] for a in range(AS)]
    Y = jnp.stack(parts, axis=0)
    q_t = jnp.transpose(Y, (4, 2, 1, 3, 0, 5))
    q_t = q_t.reshape(OW, DIM)
    q_ref[...] = q_t
    qsq_ref[...] = jnp.sum(q_t * q_t, axis=1, keepdims=True)


def _search_kernel(q_ref, qsq_ref, k_ref, dist_ref, idx_ref):
    ki = pl.program_id(0)
    kt = k_ref[...]
    ksq = jnp.sum(kt * kt, axis=1)[None, :]
    dot = jax.lax.dot_general(
        q_ref[...], kt,
        dimension_numbers=(((1,), (1,)), ((), ())),
        precision=jax.lax.Precision.HIGHEST,
        preferred_element_type=jnp.float32,
    )
    d = (qsq_ref[...] + ksq) - 2.0 * dot
    m = jnp.min(d, axis=1, keepdims=True)
    iota = jax.lax.broadcasted_iota(jnp.int32, d.shape, 1) + ki * TK
    am = jnp.min(jnp.where(d == m, iota, jnp.int32(2**30)), axis=1,
                 keepdims=True)

    @pl.when(ki == 0)
    def _init():
        dist_ref[...] = m
        idx_ref[...] = am

    @pl.when(ki != 0)
    def _update():
        better = m < dist_ref[...]
        idx_ref[...] = jnp.where(better, am, idx_ref[...])
        dist_ref[...] = jnp.where(better, m, dist_ref[...])


@functools.partial(jax.jit, static_argnames=("interpret",))
def kernel(image, mem_keys, interpret=False):
    img = jnp.transpose(image, (2, 0, 1))
    img = jnp.pad(img, ((0, 0), (PAD, PAD), (PAD, PAD)))
    img4 = img.reshape(C, U, STRIDE, H + 2 * PAD).transpose(1, 0, 2, 3)

    q, qsq = pl.pallas_call(
        _unfold_kernel,
        grid=(OH,),
        in_specs=[pl.BlockSpec(img4.shape, lambda i: (0, 0, 0, 0))],
        out_specs=[
            pl.BlockSpec((OW, DIM), lambda i: (i, 0)),
            pl.BlockSpec((OW, 1), lambda i: (i, 0)),
        ],
        out_shape=[
            jax.ShapeDtypeStruct((Q, DIM), jnp.float32),
            jax.ShapeDtypeStruct((Q, 1), jnp.float32),
        ],
        interpret=interpret,
    )(img4)

    dists, idx = pl.pallas_call(
        _search_kernel,
        grid=(N_MEM // TK,),
        in_specs=[
            pl.BlockSpec((Q, DIM), lambda ki: (0, 0)),
            pl.BlockSpec((Q, 1), lambda ki: (0, 0)),
            pl.BlockSpec((TK, DIM), lambda ki: (ki, 0)),
        ],
        out_specs=[
            pl.BlockSpec((Q, 1), lambda ki: (0, 0)),
            pl.BlockSpec((Q, 1), lambda ki: (0, 0)),
        ],
        out_shape=[
            jax.ShapeDtypeStruct((Q, 1), jnp.float32),
            jax.ShapeDtypeStruct((Q, 1), jnp.int32),
        ],
        interpret=interpret,
    )(q, qsq, mem_keys)

    return dists[:, 0], idx[:, 0]


# speedup vs baseline: 198.7203x; 198.7203x over previous
import functools

import jax
import jax.numpy as jnp
from jax.experimental import pallas as pl

KH, KW = 32, 32
STRIDE = 4
PAD = 10
H, W, C = 224, 224, 3
DIM = KH * KW * C
N_MEM = 4096
OH = (H + 2 * PAD - KH) // STRIDE + 1
OW = (W + 2 * PAD - KW) // STRIDE + 1
Q = OH * OW
U = (H + 2 * PAD) // STRIDE
AR = KH // STRIDE
OWP = 56
QP = OH * OWP

TQ = 1008
TK = 256


def _unfold_kernel(img_ref, q_ref, qsq_ref):
    i = pl.program_id(0)
    blk = img_ref[:, pl.ds(i, AR), :, :]
    rm = blk.reshape(C * KH, H + 2 * PAD)
    parts = [rm[:, 4 * min(j, OW - 1) : 4 * min(j, OW - 1) + KW]
             for j in range(OWP)]
    q3 = jnp.stack(parts, axis=0)
    q_t = q3.reshape(OWP, DIM)
    q_ref[0] = q_t
    qsq_ref[0] = jnp.sum(q_t * q_t, axis=1, keepdims=True)


def _search_kernel(q_ref, qsq_ref, k_ref, dist_ref, idx_ref):
    ki = pl.program_id(1)
    kt = k_ref[...]
    ksq = jnp.sum(kt * kt, axis=1)[None, :]
    dot = jax.lax.dot_general(
        q_ref[...], kt,
        dimension_numbers=(((1,), (1,)), ((), ())),
        precision=jax.lax.Precision.DEFAULT,
        preferred_element_type=jnp.float32,
    )
    d = (qsq_ref[...] + ksq) - 2.0 * dot
    m = jnp.min(d, axis=1, keepdims=True)
    iota = jax.lax.broadcasted_iota(jnp.int32, d.shape, 1) + ki * TK
    am = jnp.min(jnp.where(d == m, iota, jnp.int32(2**30)), axis=1,
                 keepdims=True)

    @pl.when(ki == 0)
    def _init():
        dist_ref[...] = m
        idx_ref[...] = am

    @pl.when(ki != 0)
    def _update():
        better = m < dist_ref[...]
        idx_ref[...] = jnp.where(better, am, idx_ref[...])
        dist_ref[...] = jnp.where(better, m, dist_ref[...])


@functools.partial(jax.jit, static_argnames=("interpret",))
def kernel(image, mem_keys, interpret=False):
    img = jnp.transpose(image, (2, 0, 1))
    img = jnp.pad(img, ((0, 0), (PAD, PAD), (PAD, PAD)))
    img4 = img.reshape(C, U, STRIDE, H + 2 * PAD)

    q, qsq = pl.pallas_call(
        _unfold_kernel,
        grid=(OH,),
        in_specs=[pl.BlockSpec(img4.shape, lambda i: (0, 0, 0, 0))],
        out_specs=[
            pl.BlockSpec((1, OWP, DIM), lambda i: (i, 0, 0)),
            pl.BlockSpec((1, OWP, 1), lambda i: (i, 0, 0)),
        ],
        out_shape=[
            jax.ShapeDtypeStruct((OH, OWP, DIM), jnp.float32),
            jax.ShapeDtypeStruct((OH, OWP, 1), jnp.float32),
        ],
        interpret=interpret,
    )(img4)
    q = q.reshape(QP, DIM)
    qsq = qsq.reshape(QP, 1)

    dists, idx = pl.pallas_call(
        _search_kernel,
        grid=(QP // TQ, N_MEM // TK),
        in_specs=[
            pl.BlockSpec((TQ, DIM), lambda qi, ki: (qi, 0)),
            pl.BlockSpec((TQ, 1), lambda qi, ki: (qi, 0)),
            pl.BlockSpec((TK, DIM), lambda qi, ki: (ki, 0)),
        ],
        out_specs=[
            pl.BlockSpec((TQ, 1), lambda qi, ki: (qi, 0)),
            pl.BlockSpec((TQ, 1), lambda qi, ki: (qi, 0)),
        ],
        out_shape=[
            jax.ShapeDtypeStruct((QP, 1), jnp.float32),
            jax.ShapeDtypeStruct((QP, 1), jnp.int32),
        ],
        interpret=interpret,
    )(q, qsq, mem_keys)

    dists = dists.reshape(OH, OWP)[:, :OW].reshape(Q)
    idx = idx.reshape(OH, OWP)[:, :OW].reshape(Q)
    return dists, idx
